# baseline (speedup 1.0000x reference)
import functools

import jax
import jax.numpy as jnp
from jax import lax
from jax.experimental import pallas as pl
from jax.experimental.pallas import tpu as pltpu
from jax.experimental.pallas import tpu_sc as plsc

NUM_FACTORS = 32
BATCH = 16384
GROUP = 128 // NUM_FACTORS
GROUP_SHIFT = GROUP.bit_length() - 1
NUM_GROUPS = 1000000 // GROUP
NC = 2
NS = 16
NW = NC * NS
BPW = BATCH // NW
CHUNK = 128
NCH = BPW // CHUNK
NBLK = CHUNK // 16

_mesh = plsc.VectorSubcoreMesh(
    core_axis_name="c", subcore_axis_name="s", num_cores=NC, num_subcores=NS
)


@functools.partial(
    pl.kernel,
    out_type=jax.ShapeDtypeStruct((BATCH,), jnp.float32),
    mesh=_mesh,
    compiler_params=pltpu.CompilerParams(needs_layout_passes=False),
    scratch_types=[
        pltpu.VMEM((NCH, CHUNK), jnp.int32),
        pltpu.VMEM((NCH, CHUNK), jnp.int32),
        pltpu.VMEM((NCH, CHUNK), jnp.int32),
        pltpu.VMEM((NCH, CHUNK), jnp.int32),
        pltpu.VMEM((2, CHUNK, 128), jnp.float32),
        pltpu.VMEM((2, CHUNK, 128), jnp.float32),
        pltpu.VMEM((BPW,), jnp.float32),
        pltpu.SemaphoreType.DMA,
        pltpu.SemaphoreType.DMA,
    ],
)
def _mf_dot(uf_hbm, if_hbm, user_hbm, item_hbm, out_hbm,
            uidx_v, iidx_v, ugrp_v, igrp_v, ubuf, ibuf, out_v, sem0, sem1):
    wid = lax.axis_index("s") * NC + lax.axis_index("c")
    base = wid * BPW
    sems = (sem0, sem1)

    pltpu.sync_copy(user_hbm.at[pl.ds(wid * NCH, NCH)], uidx_v)
    pltpu.sync_copy(item_hbm.at[pl.ds(wid * NCH, NCH)], iidx_v)

    for j in range(NCH):
        for v in range(CHUNK // 16):
            sl = pl.ds(v * 16, 16)
            ugrp_v[j, sl] = uidx_v[j, sl] >> GROUP_SHIFT
            igrp_v[j, sl] = iidx_v[j, sl] >> GROUP_SHIFT

    def fire(j):
        s = j & 1
        return (
            pltpu.async_copy(uf_hbm.at[ugrp_v.at[j]], ubuf.at[s], sems[s]),
            pltpu.async_copy(if_hbm.at[igrp_v.at[j]], ibuf.at[s], sems[s]),
        )

    iota16 = lax.iota(jnp.int32, 16)
    pending = fire(0)
    for j in range(NCH):
        nxt = fire(j + 1) if j + 1 < NCH else None
        for cp in pending:
            cp.wait()
        pending = nxt

        s = j & 1
        ub, ib = ubuf.at[s], ibuf.at[s]

        def block(bb, _, j=j, ub=ub, ib=ib):
            sl = pl.ds(pl.multiple_of(bb * 16, 16), 16)
            rows = bb * 16 + iota16
            ucol = (uidx_v[j, sl] & (GROUP - 1)) * NUM_FACTORS
            icol = (iidx_v[j, sl] & (GROUP - 1)) * NUM_FACTORS
            acc = jnp.zeros((16,), jnp.float32)
            for f in range(NUM_FACTORS):
                u = plsc.load_gather(ub, [rows, ucol + f])
                it = plsc.load_gather(ib, [rows, icol + f])
                acc = acc + u * it
            out_v[pl.ds(pl.multiple_of(j * CHUNK + bb * 16, 16), 16)] = acc
            return 0

        lax.fori_loop(0, NBLK, block, 0)

    pltpu.sync_copy(out_v, out_hbm.at[pl.ds(base, BPW)])


def kernel(user_factors, item_factors, user, item):
    uf_g = user_factors.reshape(NUM_GROUPS, 128)
    if_g = item_factors.reshape(NUM_GROUPS, 128)
    user_c = user.astype(jnp.int32).reshape(NW * NCH, CHUNK)
    item_c = item.astype(jnp.int32).reshape(NW * NCH, CHUNK)
    return _mf_dot(uf_g, if_g, user_c, item_c)


# speedup vs baseline: 3.9773x; 3.9773x over previous
import functools

import jax
import jax.numpy as jnp
from jax import lax
from jax.experimental import pallas as pl
from jax.experimental.pallas import tpu as pltpu
from jax.experimental.pallas import tpu_sc as plsc

NUM_ROWS = 1000000
NUM_FACTORS = 32
BATCH = 16384
NC = 2
NS = 16
NW = NC * NS
BPW = BATCH // NW
NSLOT = 8
GRP = 16
NGRP = BPW // GRP

_mesh = plsc.VectorSubcoreMesh(
    core_axis_name="c", subcore_axis_name="s", num_cores=NC, num_subcores=NS
)


@functools.partial(
    pl.kernel,
    out_type=jax.ShapeDtypeStruct((BATCH,), jnp.float32),
    mesh=_mesh,
    compiler_params=pltpu.CompilerParams(needs_layout_passes=False),
    scratch_types=[
        pltpu.VMEM((BPW,), jnp.int32),
        pltpu.VMEM((BPW,), jnp.int32),
        pltpu.VMEM((NSLOT * NUM_FACTORS, 128), jnp.float32),
        pltpu.VMEM((NSLOT * NUM_FACTORS, 128), jnp.float32),
        pltpu.VMEM((GRP, 16), jnp.float32),
        pltpu.VMEM((BPW,), jnp.float32),
    ]
    + [pltpu.SemaphoreType.DMA] * (2 * NSLOT),
)
def _mf_dot(uf_t, if_t, user_hbm, item_hbm, out_hbm,
            us_s, is_s, ubuf, ibuf, part_v, out_v, *sems):
    usem, isem = sems[:NSLOT], sems[NSLOT:]
    wid = lax.axis_index("s") * NC + lax.axis_index("c")
    base = wid * BPW

    pltpu.sync_copy(user_hbm.at[pl.ds(base, BPW)], us_s)
    pltpu.sync_copy(item_hbm.at[pl.ds(base, BPW)], is_s)

    iota16 = lax.iota(jnp.int32, 16)

    def fire(ru, ri, slot):
        uc0 = pl.multiple_of((ru >> 7) << 7, 128)
        ic0 = pl.multiple_of((ri >> 7) << 7, 128)
        pltpu.async_copy(
            uf_t.at[:, pl.ds(uc0, 128)],
            ubuf.at[pl.ds(slot * NUM_FACTORS, NUM_FACTORS)], usem[slot])
        pltpu.async_copy(
            if_t.at[:, pl.ds(ic0, 128)],
            ibuf.at[pl.ds(slot * NUM_FACTORS, NUM_FACTORS)], isem[slot])

    def drain(slot):
        pltpu.make_async_copy(
            uf_t.at[:, pl.ds(0, 128)],
            ubuf.at[pl.ds(slot * NUM_FACTORS, NUM_FACTORS)], usem[slot]).wait()
        pltpu.make_async_copy(
            if_t.at[:, pl.ds(0, 128)],
            ibuf.at[pl.ds(slot * NUM_FACTORS, NUM_FACTORS)], isem[slot]).wait()

    def item(uvec, ivec, j, slot, prow):
        drain(slot)
        ulane = jnp.full((16,), uvec[j] & 127, jnp.int32)
        ilane = jnp.full((16,), ivec[j] & 127, jnp.int32)
        rbase = slot * NUM_FACTORS
        u_lo = plsc.load_gather(ubuf, [rbase + iota16, ulane])
        u_hi = plsc.load_gather(ubuf, [rbase + 16 + iota16, ulane])
        i_lo = plsc.load_gather(ibuf, [rbase + iota16, ilane])
        i_hi = plsc.load_gather(ibuf, [rbase + 16 + iota16, ilane])
        part_v[prow] = u_lo * i_lo + u_hi * i_hi

    uvec0 = us_s[pl.ds(0, 16)]
    ivec0 = is_s[pl.ds(0, 16)]
    for j in range(NSLOT):
        fire(uvec0[j], ivec0[j], j)

    def group(g, _):
        uvec = us_s[pl.ds(pl.multiple_of(g * GRP, 16), 16)]
        ivec = is_s[pl.ds(pl.multiple_of(g * GRP, 16), 16)]
        nxt = pl.multiple_of((g + 1) * GRP, 16)
        nxt = pl.multiple_of(jnp.minimum(nxt, BPW - 16), 16)
        uvecn = us_s[pl.ds(nxt, 16)]
        ivecn = is_s[pl.ds(nxt, 16)]

        for j in range(NSLOT):
            item(uvec, ivec, j, j, j)
            fire(uvec[NSLOT + j], ivec[NSLOT + j], j)
        for j in range(NSLOT):
            item(uvec, ivec, NSLOT + j, j, NSLOT + j)

            @pl.when(g + 1 < NGRP)
            def _(j=j):
                fire(uvecn[j], ivecn[j], j)

        acc = jnp.zeros((16,), jnp.float32)
        for f in range(16):
            acc = acc + plsc.load_gather(
                part_v, [iota16, jnp.full((16,), f, jnp.int32)])
        out_v[pl.ds(pl.multiple_of(g * GRP, 16), GRP)] = acc
        return 0

    lax.fori_loop(0, NGRP, group, 0)
    pltpu.sync_copy(out_v, out_hbm.at[pl.ds(base, BPW)])


def kernel(user_factors, item_factors, user, item):
    return _mf_dot(user_factors.T, item_factors.T,
                   user.astype(jnp.int32), item.astype(jnp.int32))
